# baseline (speedup 1.0000x reference)
import functools

import jax
import jax.numpy as jnp
from jax import lax
from jax.experimental import pallas as pl
from jax.experimental.pallas import tpu as pltpu
from jax.experimental.pallas import tpu_sc as plsc

NC = 2
NS = 16
NW = NC * NS
CH = 128


def _split(tch):
    base = (tch // NW) // 2 * 2
    npairs = (tch - base * NW) // 2
    return base, npairs


def _worker_range(wid, base, npairs):
    t = base + 2 * (wid < npairs).astype(jnp.int32)
    start = base * wid + 2 * jnp.minimum(wid, npairs)
    return start, t


@functools.lru_cache(maxsize=None)
def _make_deg_kernel(tch, npad):
    base, npairs = _split(tch)
    mb = base + 2
    rpt = npad // NS
    mesh = plsc.VectorSubcoreMesh(core_axis_name="c", subcore_axis_name="s",
                                  num_cores=NC, num_subcores=NS)

    @functools.partial(
        pl.kernel,
        out_type=jax.ShapeDtypeStruct((NC, npad), jnp.float32),
        mesh=mesh,
        compiler_params=pltpu.CompilerParams(use_tc_tiling_on_sc=False),
        scratch_types=[
            pltpu.VMEM((mb, CH), jnp.int32),
            pltpu.VMEM((CH,), jnp.float32),
            pltpu.VMEM((rpt,), jnp.float32),
            pltpu.VMEM_SHARED((npad,), jnp.float32),
        ],
    )
    def deg_kernel(dst_hbm, out_hbm, dst_v, ones_v, z_v, acc_sh):
        cid = lax.axis_index("c")
        sid = lax.axis_index("s")
        wid = cid * NS + sid
        start, t = _worker_range(wid, base, npairs)
        pltpu.sync_copy(dst_hbm.at[pl.ds(start, base)],
                        dst_v.at[pl.ds(0, base)])
        st2 = jnp.minimum(start + base, tch - 2)
        pltpu.sync_copy(dst_hbm.at[pl.ds(st2, 2)], dst_v.at[pl.ds(base, 2)])
        for i in range(CH // 16):
            ones_v[pl.ds(i * 16, 16)] = jnp.ones((16,), jnp.float32)

        def zero_body(i, c):
            z_v[pl.ds(i * 16, 16)] = jnp.zeros((16,), jnp.float32)
            return c

        lax.fori_loop(0, rpt // 16, zero_body, 0)
        pltpu.sync_copy(z_v, acc_sh.at[pl.ds(sid * rpt, rpt)])
        plsc.subcore_barrier()

        def body(j, c):
            pltpu.sync_copy(ones_v, acc_sh.at[dst_v.at[j]], add=True)
            return c

        lax.fori_loop(0, t, body, 0)
        plsc.subcore_barrier()
        pltpu.sync_copy(acc_sh.at[pl.ds(sid * rpt, rpt)],
                        out_hbm.at[cid, pl.ds(sid * rpt, rpt)])

    return deg_kernel


@functools.lru_cache(maxsize=None)
def _make_edge_kernel(tch, npad, feat, n_g):
    base, npairs = _split(tch)
    mb = base + 2
    rpt = npad // NS
    rpt_g = -(-n_g // NS) // 8 * 8 + 8
    mesh = plsc.VectorSubcoreMesh(core_axis_name="c", subcore_axis_name="s",
                                  num_cores=NC, num_subcores=NS)

    @functools.partial(
        pl.kernel,
        out_type=jax.ShapeDtypeStruct((NC, npad, feat), jnp.float32),
        mesh=mesh,
        compiler_params=pltpu.CompilerParams(use_tc_tiling_on_sc=False),
        scratch_types=[
            pltpu.VMEM((mb, CH), jnp.int32),
            pltpu.VMEM((mb, CH), jnp.int32),
            pltpu.VMEM((2, CH, feat), jnp.float32),
            pltpu.VMEM((rpt, feat), jnp.float32),
            pltpu.VMEM_SHARED((npad, feat), jnp.float32),
            pltpu.VMEM_SHARED((n_g, feat), jnp.float32),
            pltpu.SemaphoreType.DMA,
            pltpu.SemaphoreType.DMA,
            pltpu.SemaphoreType.DMA,
            pltpu.SemaphoreType.DMA,
        ],
    )
    def edge_kernel(src_hbm, dst_hbm, g_hbm, out_hbm,
                    src_v, dst_v, rows_v, z_v, acc_sh, g_sh,
                    gsem0, gsem1, ssem0, ssem1):
        cid = lax.axis_index("c")
        sid = lax.axis_index("s")
        wid = cid * NS + sid
        start, t = _worker_range(wid, base, npairs)
        pltpu.sync_copy(src_hbm.at[pl.ds(start, base)],
                        src_v.at[pl.ds(0, base)])
        pltpu.sync_copy(dst_hbm.at[pl.ds(start, base)],
                        dst_v.at[pl.ds(0, base)])
        st2 = jnp.minimum(start + base, tch - 2)
        pltpu.sync_copy(src_hbm.at[pl.ds(st2, 2)], src_v.at[pl.ds(base, 2)])
        pltpu.sync_copy(dst_hbm.at[pl.ds(st2, 2)], dst_v.at[pl.ds(base, 2)])
        gbase = jnp.minimum(sid * rpt_g, n_g - rpt_g)
        pltpu.sync_copy(g_hbm.at[pl.ds(gbase, rpt_g)],
                        g_sh.at[pl.ds(gbase, rpt_g)])

        def zero_body(i, c):
            z_v[i] = jnp.zeros((feat,), jnp.float32)
            return c

        lax.fori_loop(0, rpt, zero_body, 0)
        pltpu.sync_copy(z_v, acc_sh.at[pl.ds(sid * rpt, rpt)])
        plsc.subcore_barrier()

        pltpu.async_copy(g_sh.at[src_v.at[0]], rows_v.at[0], gsem0)
        pltpu.async_copy(g_sh.at[src_v.at[1]], rows_v.at[1], gsem1)

        def body(i, c):
            j0 = 2 * i
            j1 = j0 + 1
            pltpu.make_async_copy(g_sh.at[src_v.at[j0]],
                                  rows_v.at[0], gsem0).wait()
            pltpu.async_copy(rows_v.at[0], acc_sh.at[dst_v.at[j0]],
                             ssem0, add=True)
            pltpu.make_async_copy(g_sh.at[src_v.at[j1]],
                                  rows_v.at[1], gsem1).wait()
            pltpu.async_copy(rows_v.at[1], acc_sh.at[dst_v.at[j1]],
                             ssem1, add=True)
            pltpu.make_async_copy(rows_v.at[0], acc_sh.at[dst_v.at[j0]],
                                  ssem0).wait()
            pltpu.async_copy(g_sh.at[src_v.at[jnp.minimum(j0 + 2, t - 2)]],
                             rows_v.at[0], gsem0)
            pltpu.make_async_copy(rows_v.at[1], acc_sh.at[dst_v.at[j1]],
                                  ssem1).wait()
            pltpu.async_copy(g_sh.at[src_v.at[jnp.minimum(j1 + 2, t - 1)]],
                             rows_v.at[1], gsem1)
            return c

        lax.fori_loop(0, t // 2, body, 0)
        pltpu.make_async_copy(g_sh.at[src_v.at[0]], rows_v.at[0],
                              gsem0).wait()
        pltpu.make_async_copy(g_sh.at[src_v.at[1]], rows_v.at[1],
                              gsem1).wait()
        plsc.subcore_barrier()
        pltpu.sync_copy(acc_sh.at[pl.ds(sid * rpt, rpt)],
                        out_hbm.at[cid, pl.ds(sid * rpt, rpt)])

    return edge_kernel


def _tc1_body(dp_ref, x_ref, w_ref, g_ref):
    deg = dp_ref[0] + dp_ref[1] + 1.0
    dinv = lax.rsqrt(deg)
    h = jnp.dot(x_ref[...], w_ref[...], preferred_element_type=jnp.float32)
    g_ref[...] = h * dinv


def _tc2_body(sp_ref, dp_ref, g1_ref, b1_ref, g2_ref):
    deg = dp_ref[0] + dp_ref[1] + 1.0
    dinv = lax.rsqrt(deg)
    s = sp_ref[0] + sp_ref[1] + g1_ref[...]
    h1 = jnp.maximum(s * dinv + b1_ref[...], 0.0)
    g2_ref[...] = h1 * dinv


def _tc3_body(sp_ref, dp_ref, g2_ref, w2_ref, b2_ref, o_ref):
    deg = dp_ref[0] + dp_ref[1] + 1.0
    dinv = lax.rsqrt(deg)
    s = sp_ref[0] + sp_ref[1] + g2_ref[...]
    t = jnp.dot(s * dinv, w2_ref[...], preferred_element_type=jnp.float32)
    o_ref[...] = jax.nn.sigmoid(t + b2_ref[...])


def kernel(x, edge_index, batch, W1, b1, W2, b2):
    n, f = x.shape
    h = W1.shape[1]
    e = edge_index.shape[1]
    npad = -(-(n + 1) // 256) * 256

    src = edge_index[0]
    dst = edge_index[1]
    if e % CH:
        pad = CH - e % CH
        src = jnp.concatenate([src, jnp.zeros((pad,), jnp.int32)])
        dst = jnp.concatenate([dst, jnp.full((pad,), n, jnp.int32)])
    tch = src.shape[0] // CH
    src2 = src.reshape(tch, CH)
    dst2 = dst.reshape(tch, CH)

    deg_k = _make_deg_kernel(tch, npad)
    edge_k = _make_edge_kernel(tch, npad, h, n)

    degp = deg_k(dst2)
    dp = degp.reshape(NC, npad, 1)

    blk = 2000
    grid = (n // blk,)
    dp_spec = pl.BlockSpec((NC, blk, 1), lambda i: (0, i, 0))
    sp_spec = pl.BlockSpec((NC, blk, h), lambda i: (0, i, 0))
    g_spec = pl.BlockSpec((blk, h), lambda i: (i, 0))

    g1 = pl.pallas_call(
        _tc1_body,
        grid=grid,
        in_specs=[dp_spec,
                  pl.BlockSpec((blk, f), lambda i: (i, 0)),
                  pl.BlockSpec((f, h), lambda i: (0, 0))],
        out_specs=g_spec,
        out_shape=jax.ShapeDtypeStruct((n, h), jnp.float32),
    )(dp, x, W1)

    s1p = edge_k(src2, dst2, g1)

    g2 = pl.pallas_call(
        _tc2_body,
        grid=grid,
        in_specs=[sp_spec, dp_spec, g_spec,
                  pl.BlockSpec((1, h), lambda i: (0, 0))],
        out_specs=g_spec,
        out_shape=jax.ShapeDtypeStruct((n, h), jnp.float32),
    )(s1p, dp, g1, b1.reshape(1, h))

    s2p = edge_k(src2, dst2, g2)

    out = pl.pallas_call(
        _tc3_body,
        grid=grid,
        in_specs=[sp_spec, dp_spec, g_spec,
                  pl.BlockSpec((h, 1), lambda i: (0, 0)),
                  pl.BlockSpec((1, 1), lambda i: (0, 0))],
        out_specs=pl.BlockSpec((blk, 1), lambda i: (i, 0)),
        out_shape=jax.ShapeDtypeStruct((n, 1), jnp.float32),
    )(s2p, dp, g2, W2, b2.reshape(1, 1))

    return out
